# baseline (speedup 1.0000x reference)
import functools

import jax
import jax.numpy as jnp
from jax import lax
from jax.experimental import pallas as pl
from jax.experimental.pallas import tpu as pltpu, tpu_sc as plsc

N_NODES = 10000
N_EDGES = 320000
D_IN = 128
HID = 128
N_CLASSES = 10
N_GRAPHS = 64
N_LAYERS = 3
BN_EPS = 1e-5

HALF = HID // 2
N_TILES = 16
EDGES_PER_TILE = N_EDGES // N_TILES
CHUNK = 80
K = 2
N_CHUNKS = 256
EDGES_PER_TILE_PAD = N_CHUNKS * CHUNK
ROWS_PER_TILE = N_NODES // N_TILES
N_PHASES = 2
CPP = N_CHUNKS // N_PHASES
NT = CPP // (2 * K)
ACC_ROWS = N_NODES + CHUNK


def _sc_body(h2, src3, dst3, z_out, h_sp, acc, src_v, dst_v, bufs,
             gsA, gsB, ssA, ssB):
    c = lax.axis_index("c")
    s = lax.axis_index("s")
    h2c = h2.at[c]

    def drain(sem, n):
        for _ in range(n):
            pltpu.make_async_copy(h2c.at[pl.ds(0, CHUNK)], bufs.at[0],
                                  sem).wait()

    rows = pl.ds(s * ROWS_PER_TILE, ROWS_PER_TILE)
    pltpu.sync_copy(h2c.at[rows], h_sp.at[rows])
    pltpu.sync_copy(h2c.at[rows], acc.at[rows])
    plsc.subcore_barrier()

    for p in range(N_PHASES):
        pchunks = pl.ds(p * CPP, CPP)
        pltpu.sync_copy(src3.at[s].at[pchunks], src_v)
        pltpu.sync_copy(dst3.at[s].at[pchunks], dst_v)

        for i in range(K):
            pltpu.async_copy(h_sp.at[src_v.at[i]], bufs.at[i], gsA)
        for i in range(K):
            pltpu.async_copy(bufs.at[K + i], acc.at[pl.ds(N_NODES, CHUNK)],
                             ssB)

        def step(t, _):
            g0 = 2 * t
            g1 = g0 + 1
            gw = lax.rem(g0 + 2, CPP)
            drain(ssB, K)
            for i in range(K):
                pltpu.async_copy(h2c.at[src_v.at[g1 * K + i]],
                                 bufs.at[K + i], gsB)
            drain(gsA, K)
            for i in range(K):
                pltpu.async_copy(bufs.at[i], acc.at[dst_v.at[g0 * K + i]],
                                 ssA, add=True)
            drain(ssA, K)
            for i in range(K):
                pltpu.async_copy(h_sp.at[src_v.at[gw * K + i]], bufs.at[i],
                                 gsA)
            drain(gsB, K)
            for i in range(K):
                pltpu.async_copy(bufs.at[K + i],
                                 acc.at[dst_v.at[g1 * K + i]], ssB, add=True)
            return _

        lax.fori_loop(0, NT, step, 0, unroll=False)
        drain(ssB, K)
        drain(gsA, K)

    plsc.subcore_barrier()

    pltpu.sync_copy(acc.at[rows], z_out.at[c].at[rows])


def _make_sc_segsum():
    mesh = plsc.VectorSubcoreMesh(core_axis_name="c", subcore_axis_name="s")
    return pl.kernel(
        _sc_body,
        out_type=jax.ShapeDtypeStruct((2, N_NODES, HALF), jnp.float32),
        mesh=mesh,
        compiler_params=pltpu.CompilerParams(use_tc_tiling_on_sc=False),
        scratch_types=[
            pltpu.VMEM_SHARED((N_NODES, HALF), jnp.float32),
            pltpu.VMEM_SHARED((ACC_ROWS, HALF), jnp.float32),
            pltpu.VMEM((CPP, CHUNK), jnp.int32),
            pltpu.VMEM((CPP, CHUNK), jnp.int32),
            pltpu.VMEM((2 * K, CHUNK, HALF), jnp.float32),
            pltpu.SemaphoreType.DMA,
            pltpu.SemaphoreType.DMA,
            pltpu.SemaphoreType.DMA,
            pltpu.SemaphoreType.DMA,
        ],
    )


def _tc_layer_body(z2, h2, w1, b1, w2, b2, gam, bet, out):
    zin = jnp.concatenate([z2[0], z2[1]], axis=-1)
    t = jnp.dot(zin, w1[...], preferred_element_type=jnp.float32) + b1[...]
    t = jnp.maximum(t, 0.0)
    t = jnp.dot(t, w2[...], preferred_element_type=jnp.float32) + b2[...]
    mu = jnp.mean(t, axis=0, keepdims=True)
    var = jnp.mean((t - mu) ** 2, axis=0, keepdims=True)
    t = (t - mu) * lax.rsqrt(var + BN_EPS) * gam[...] + bet[...]
    t = jnp.maximum(t, 0.0)
    hnew = jnp.concatenate([h2[0], h2[1]], axis=-1) + t
    out[0] = hnew[:, :HALF]
    out[1] = hnew[:, HALF:]


def _tc_layer(z2, h2, p):
    return pl.pallas_call(
        _tc_layer_body,
        out_shape=jax.ShapeDtypeStruct((2, N_NODES, HALF), jnp.float32),
    )(z2, h2, p["W1"], p["b1"][None, :], p["W2"], p["b2"][None, :],
      p["gamma"][None, :], p["beta"][None, :])


def _tc_pool_body(h2, batch, wm1, bm1, wm2, bm2, out):
    h = jnp.concatenate([h2[0], h2[1]], axis=-1)
    seg = lax.broadcasted_iota(jnp.int32, (N_GRAPHS, N_NODES), 0)
    onehot = (seg == batch[...]).astype(jnp.float32)
    sums = jnp.dot(onehot, h, preferred_element_type=jnp.float32)
    counts = jnp.sum(onehot, axis=1, keepdims=True)
    g = sums / jnp.maximum(counts, 1.0)
    g = jnp.maximum(
        jnp.dot(g, wm1[...], preferred_element_type=jnp.float32) + bm1[...],
        0.0)
    out[...] = jnp.dot(g, wm2[...], preferred_element_type=jnp.float32) \
        + bm2[...]


def _tc_pool(h2, batch2d, params):
    return pl.pallas_call(
        _tc_pool_body,
        out_shape=jax.ShapeDtypeStruct((N_GRAPHS, N_CLASSES), jnp.float32),
    )(h2, batch2d, params["Wm1"], params["bm1"][None, :],
      params["Wm2"], params["bm2"][None, :])


def kernel(x, edge_index, batch, params):
    n_pad = EDGES_PER_TILE_PAD - EDGES_PER_TILE
    src3 = jnp.concatenate(
        [edge_index[0].reshape(N_TILES, EDGES_PER_TILE),
         jnp.zeros((N_TILES, n_pad), jnp.int32)],
        axis=1).reshape(N_TILES, N_CHUNKS, CHUNK)
    dst3 = jnp.concatenate(
        [edge_index[1].reshape(N_TILES, EDGES_PER_TILE),
         jnp.full((N_TILES, n_pad), N_NODES, jnp.int32)],
        axis=1).reshape(N_TILES, N_CHUNKS, CHUNK)
    h2 = x.reshape(N_NODES, 2, HALF).transpose(1, 0, 2)
    sc_segsum = _make_sc_segsum()
    for l in range(N_LAYERS):
        z2 = sc_segsum(h2, src3, dst3)
        h2 = _tc_layer(z2, h2, params[f"layer{l}"])
    return _tc_pool(h2, batch[None, :], params)


# speedup vs baseline: 1.3583x; 1.3583x over previous
import functools

import jax
import jax.numpy as jnp
from jax import lax
from jax.experimental import pallas as pl
from jax.experimental.pallas import tpu as pltpu, tpu_sc as plsc

N_NODES = 10000
N_EDGES = 320000
D_IN = 128
HID = 128
N_CLASSES = 10
N_GRAPHS = 64
N_LAYERS = 3
BN_EPS = 1e-5

HALF = HID // 2
N_TILES = 16
EDGES_PER_TILE = N_EDGES // N_TILES
CHUNK = 80
K = 2
N_CHUNKS = 256
EDGES_PER_TILE_PAD = N_CHUNKS * CHUNK
ROWS_PER_TILE = N_NODES // N_TILES
N_PHASES = 2
CPP = N_CHUNKS // N_PHASES
NT = CPP // (2 * K)
ACC_ROWS = N_NODES + CHUNK


def _sc_body(h2, src3, dst3, z_out, h_sp, acc, src_v, dst_v, bufs,
             gsA, gsB, ssA, ssB):
    c = lax.axis_index("c")
    s = lax.axis_index("s")
    h2c = h2.at[c]

    def drain(sem, n):
        for _ in range(n):
            pltpu.make_async_copy(h2c.at[pl.ds(0, CHUNK)], bufs.at[0],
                                  sem).wait()

    rows = pl.ds(s * ROWS_PER_TILE, ROWS_PER_TILE)
    pltpu.sync_copy(h2c.at[rows], h_sp.at[rows])
    pltpu.sync_copy(h2c.at[rows], acc.at[rows])
    plsc.subcore_barrier()

    for p in range(N_PHASES):
        pchunks = pl.ds(p * CPP, CPP)
        pltpu.sync_copy(src3.at[s].at[pchunks], src_v)
        pltpu.sync_copy(dst3.at[s].at[pchunks], dst_v)

        for i in range(K):
            pltpu.async_copy(h_sp.at[src_v.at[i]], bufs.at[i], gsA)
        for i in range(K):
            pltpu.async_copy(bufs.at[K + i], acc.at[pl.ds(N_NODES, CHUNK)],
                             ssB)

        def step(t, _):
            g0 = 2 * t
            g1 = g0 + 1
            gw = lax.rem(g0 + 2, CPP)

            drain(ssB, K)
            for i in range(K):
                pltpu.async_copy(h_sp.at[src_v.at[g1 * K + i]],
                                 bufs.at[K + i], gsB)
            drain(gsA, K)
            for i in range(K):
                pltpu.async_copy(bufs.at[i], acc.at[dst_v.at[g0 * K + i]],
                                 ssA, add=True)
            drain(ssA, K)
            for i in range(K):
                pltpu.async_copy(h_sp.at[src_v.at[gw * K + i]], bufs.at[i],
                                 gsA)
            drain(gsB, K)
            for i in range(K):
                pltpu.async_copy(bufs.at[K + i],
                                 acc.at[dst_v.at[g1 * K + i]], ssB, add=True)
            return _

        lax.fori_loop(0, NT, step, 0, unroll=False)
        drain(ssB, K)
        drain(gsA, K)

    plsc.subcore_barrier()

    pltpu.sync_copy(acc.at[rows], z_out.at[c].at[rows])


def _make_sc_segsum():
    mesh = plsc.VectorSubcoreMesh(core_axis_name="c", subcore_axis_name="s")
    return pl.kernel(
        _sc_body,
        out_type=jax.ShapeDtypeStruct((2, N_NODES, HALF), jnp.float32),
        mesh=mesh,
        compiler_params=pltpu.CompilerParams(use_tc_tiling_on_sc=False),
        scratch_types=[
            pltpu.VMEM_SHARED((N_NODES, HALF), jnp.float32),
            pltpu.VMEM_SHARED((ACC_ROWS, HALF), jnp.float32),
            pltpu.VMEM((CPP, CHUNK), jnp.int32),
            pltpu.VMEM((CPP, CHUNK), jnp.int32),
            pltpu.VMEM((2 * K, CHUNK, HALF), jnp.float32),
            pltpu.SemaphoreType.DMA,
            pltpu.SemaphoreType.DMA,
            pltpu.SemaphoreType.DMA,
            pltpu.SemaphoreType.DMA,
        ],
    )


def _tc_layer_body(z2, h2, w1, b1, w2, b2, gam, bet, out):
    zin = jnp.concatenate([z2[0], z2[1]], axis=-1)
    t = jnp.dot(zin, w1[...], preferred_element_type=jnp.float32) + b1[...]
    t = jnp.maximum(t, 0.0)
    t = jnp.dot(t, w2[...], preferred_element_type=jnp.float32) + b2[...]
    mu = jnp.mean(t, axis=0, keepdims=True)
    var = jnp.mean((t - mu) ** 2, axis=0, keepdims=True)
    t = (t - mu) * lax.rsqrt(var + BN_EPS) * gam[...] + bet[...]
    t = jnp.maximum(t, 0.0)
    hnew = jnp.concatenate([h2[0], h2[1]], axis=-1) + t
    out[0] = hnew[:, :HALF]
    out[1] = hnew[:, HALF:]


def _tc_layer(z2, h2, p):
    return pl.pallas_call(
        _tc_layer_body,
        out_shape=jax.ShapeDtypeStruct((2, N_NODES, HALF), jnp.float32),
    )(z2, h2, p["W1"], p["b1"][None, :], p["W2"], p["b2"][None, :],
      p["gamma"][None, :], p["beta"][None, :])


def _tc_pool_body(h2, batch, wm1, bm1, wm2, bm2, out):
    h = jnp.concatenate([h2[0], h2[1]], axis=-1)
    seg = lax.broadcasted_iota(jnp.int32, (N_GRAPHS, N_NODES), 0)
    onehot = (seg == batch[...]).astype(jnp.float32)
    sums = jnp.dot(onehot, h, preferred_element_type=jnp.float32)
    counts = jnp.sum(onehot, axis=1, keepdims=True)
    g = sums / jnp.maximum(counts, 1.0)
    g = jnp.maximum(
        jnp.dot(g, wm1[...], preferred_element_type=jnp.float32) + bm1[...],
        0.0)
    out[...] = jnp.dot(g, wm2[...], preferred_element_type=jnp.float32) \
        + bm2[...]


def _tc_pool(h2, batch2d, params):
    return pl.pallas_call(
        _tc_pool_body,
        out_shape=jax.ShapeDtypeStruct((N_GRAPHS, N_CLASSES), jnp.float32),
    )(h2, batch2d, params["Wm1"], params["bm1"][None, :],
      params["Wm2"], params["bm2"][None, :])


def kernel(x, edge_index, batch, params):
    n_pad = EDGES_PER_TILE_PAD - EDGES_PER_TILE
    src3 = jnp.concatenate(
        [edge_index[0].reshape(N_TILES, EDGES_PER_TILE),
         jnp.zeros((N_TILES, n_pad), jnp.int32)],
        axis=1).reshape(N_TILES, N_CHUNKS, CHUNK)
    dst3 = jnp.concatenate(
        [edge_index[1].reshape(N_TILES, EDGES_PER_TILE),
         jnp.full((N_TILES, n_pad), N_NODES, jnp.int32)],
        axis=1).reshape(N_TILES, N_CHUNKS, CHUNK)
    h2 = x.reshape(N_NODES, 2, HALF).transpose(1, 0, 2)
    sc_segsum = _make_sc_segsum()
    for l in range(N_LAYERS):
        z2 = sc_segsum(h2, src3, dst3)
        h2 = _tc_layer(z2, h2, params[f"layer{l}"])
    return _tc_pool(h2, batch[None, :], params)


# speedup vs baseline: 1.3928x; 1.0254x over previous
import functools

import jax
import jax.numpy as jnp
from jax import lax
from jax.experimental import pallas as pl
from jax.experimental.pallas import tpu as pltpu, tpu_sc as plsc

N_NODES = 10000
N_EDGES = 320000
D_IN = 128
HID = 128
N_CLASSES = 10
N_GRAPHS = 64
N_LAYERS = 3
BN_EPS = 1e-5

HALF = HID // 2
N_TILES = 16
EDGES_PER_TILE = N_EDGES // N_TILES
CHUNK = 80
K = 2
N_CHUNKS = 256
EDGES_PER_TILE_PAD = N_CHUNKS * CHUNK
ROWS_PER_TILE = N_NODES // N_TILES
N_PHASES = 2
CPP = N_CHUNKS // N_PHASES
NT = CPP // (2 * K)
ACC_ROWS = N_NODES + CHUNK


def _sc_body(h2, src3, dst3, z_out, h_sp, acc, src_v, dst_v, bufs,
             gsA, gsB, ssA, ssB):
    c = lax.axis_index("c")
    s = lax.axis_index("s")
    h2c = h2.at[c]

    def drain(sem, n):
        for _ in range(n):
            pltpu.make_async_copy(h2c.at[pl.ds(0, CHUNK)], bufs.at[0],
                                  sem).wait()

    rows = pl.ds(s * ROWS_PER_TILE, ROWS_PER_TILE)
    pltpu.sync_copy(h2c.at[rows], h_sp.at[rows])
    pltpu.sync_copy(h2c.at[rows], acc.at[rows])
    plsc.subcore_barrier()

    for p in range(N_PHASES):
        pchunks = pl.ds(p * CPP, CPP)
        pltpu.sync_copy(src3.at[s].at[pchunks], src_v)
        pltpu.sync_copy(dst3.at[s].at[pchunks], dst_v)

        for i in range(K):
            pltpu.async_copy(h_sp.at[src_v.at[i]], bufs.at[i], gsA)
        for i in range(K):
            pltpu.async_copy(bufs.at[K + i], acc.at[pl.ds(N_NODES, CHUNK)],
                             ssB)

        def step(t, _):
            g0 = 2 * t
            g1 = g0 + 1
            gw = lax.rem(g0 + 2, CPP)

            drain(ssB, K)
            for i in range(K):
                pltpu.async_copy(h_sp.at[src_v.at[g1 * K + i]],
                                 bufs.at[K + i], gsB)
            drain(gsA, K)
            for i in range(K):
                pltpu.async_copy(bufs.at[i], acc.at[dst_v.at[g0 * K + i]],
                                 ssA, add=True)
            drain(ssA, K)
            for i in range(K):
                pltpu.async_copy(h_sp.at[src_v.at[gw * K + i]], bufs.at[i],
                                 gsA)
            drain(gsB, K)
            for i in range(K):
                pltpu.async_copy(bufs.at[K + i],
                                 acc.at[dst_v.at[g1 * K + i]], ssB, add=True)
            return _

        lax.fori_loop(0, NT, step, 0, unroll=False)
        drain(ssB, K)
        drain(gsA, K)

    plsc.subcore_barrier()

    pltpu.sync_copy(acc.at[rows], z_out.at[c].at[rows])


def _make_sc_segsum():
    mesh = plsc.VectorSubcoreMesh(core_axis_name="c", subcore_axis_name="s")
    return pl.kernel(
        _sc_body,
        out_type=jax.ShapeDtypeStruct((2, N_NODES, HALF), jnp.float32),
        mesh=mesh,
        compiler_params=pltpu.CompilerParams(use_tc_tiling_on_sc=False),
        scratch_types=[
            pltpu.VMEM_SHARED((N_NODES, HALF), jnp.float32),
            pltpu.VMEM_SHARED((ACC_ROWS, HALF), jnp.float32),
            pltpu.VMEM((CPP, CHUNK), jnp.int32),
            pltpu.VMEM((CPP, CHUNK), jnp.int32),
            pltpu.VMEM((2 * K, CHUNK, HALF), jnp.float32),
            pltpu.SemaphoreType.DMA,
            pltpu.SemaphoreType.DMA,
            pltpu.SemaphoreType.DMA,
            pltpu.SemaphoreType.DMA,
        ],
    )


def _dense_layer(z2, h2, w1, b1, w2, b2, gam, bet):
    zin = jnp.concatenate([z2[0], z2[1]], axis=-1)
    t = jnp.dot(zin, w1[...], preferred_element_type=jnp.float32) + b1[...]
    t = jnp.maximum(t, 0.0)
    t = jnp.dot(t, w2[...], preferred_element_type=jnp.float32) + b2[...]
    mu = jnp.mean(t, axis=0, keepdims=True)
    var = jnp.mean((t - mu) ** 2, axis=0, keepdims=True)
    t = (t - mu) * lax.rsqrt(var + BN_EPS) * gam[...] + bet[...]
    t = jnp.maximum(t, 0.0)
    return jnp.concatenate([h2[0], h2[1]], axis=-1) + t


def _tc_layer_body(z2, h2, w1, b1, w2, b2, gam, bet, out):
    hnew = _dense_layer(z2, h2, w1, b1, w2, b2, gam, bet)
    out[0] = hnew[:, :HALF]
    out[1] = hnew[:, HALF:]


def _tc_layer(z2, h2, p):
    return pl.pallas_call(
        _tc_layer_body,
        out_shape=jax.ShapeDtypeStruct((2, N_NODES, HALF), jnp.float32),
    )(z2, h2, p["W1"], p["b1"][None, :], p["W2"], p["b2"][None, :],
      p["gamma"][None, :], p["beta"][None, :])


def _tc_last_body(z2, h2, w1, b1, w2, b2, gam, bet, batch,
                  wm1, bm1, wm2, bm2, out):
    h = _dense_layer(z2, h2, w1, b1, w2, b2, gam, bet)
    seg = lax.broadcasted_iota(jnp.int32, (N_GRAPHS, N_NODES), 0)
    onehot = (seg == batch[...]).astype(jnp.float32)
    sums = jnp.dot(onehot, h, preferred_element_type=jnp.float32)
    counts = jnp.sum(onehot, axis=1, keepdims=True)
    g = sums / jnp.maximum(counts, 1.0)
    g = jnp.maximum(
        jnp.dot(g, wm1[...], preferred_element_type=jnp.float32) + bm1[...],
        0.0)
    out[...] = jnp.dot(g, wm2[...], preferred_element_type=jnp.float32) \
        + bm2[...]


def _tc_last(z2, h2, p, batch2d, params):
    return pl.pallas_call(
        _tc_last_body,
        out_shape=jax.ShapeDtypeStruct((N_GRAPHS, N_CLASSES), jnp.float32),
    )(z2, h2, p["W1"], p["b1"][None, :], p["W2"], p["b2"][None, :],
      p["gamma"][None, :], p["beta"][None, :], batch2d,
      params["Wm1"], params["bm1"][None, :],
      params["Wm2"], params["bm2"][None, :])


def kernel(x, edge_index, batch, params):
    n_pad = EDGES_PER_TILE_PAD - EDGES_PER_TILE
    src3 = jnp.concatenate(
        [edge_index[0].reshape(N_TILES, EDGES_PER_TILE),
         jnp.zeros((N_TILES, n_pad), jnp.int32)],
        axis=1).reshape(N_TILES, N_CHUNKS, CHUNK)
    dst3 = jnp.concatenate(
        [edge_index[1].reshape(N_TILES, EDGES_PER_TILE),
         jnp.full((N_TILES, n_pad), N_NODES, jnp.int32)],
        axis=1).reshape(N_TILES, N_CHUNKS, CHUNK)
    h2 = x.reshape(N_NODES, 2, HALF).transpose(1, 0, 2)
    sc_segsum = _make_sc_segsum()
    for l in range(N_LAYERS - 1):
        z2 = sc_segsum(h2, src3, dst3)
        h2 = _tc_layer(z2, h2, params[f"layer{l}"])
    z2 = sc_segsum(h2, src3, dst3)
    return _tc_last(z2, h2, params[f"layer{N_LAYERS - 1}"],
                    batch[None, :], params)


# speedup vs baseline: 1.3929x; 1.0001x over previous
import functools

import jax
import jax.numpy as jnp
from jax import lax
from jax.experimental import pallas as pl
from jax.experimental.pallas import tpu as pltpu, tpu_sc as plsc

N_NODES = 10000
N_EDGES = 320000
D_IN = 128
HID = 128
N_CLASSES = 10
N_GRAPHS = 64
N_LAYERS = 3
BN_EPS = 1e-5

HALF = HID // 2
N_TILES = 16
EDGES_PER_TILE = N_EDGES // N_TILES
CHUNK = 80
K = 2
N_CHUNKS = 256
EDGES_PER_TILE_PAD = N_CHUNKS * CHUNK
ROWS_PER_TILE = N_NODES // N_TILES
N_PHASES = 2
CPP = N_CHUNKS // N_PHASES
NT = CPP // (2 * K)
ACC_ROWS = N_NODES + CHUNK


def _sc_body(h2, src3, dst3, z_out, h_sp, acc, src_v, dst_v, bufs,
             gsA, gsB, ssA, ssB):
    c = lax.axis_index("c")
    s = lax.axis_index("s")
    h2c = h2.at[c]

    def drain(sem, n):
        for _ in range(n):
            pltpu.make_async_copy(h2c.at[pl.ds(0, CHUNK)], bufs.at[0],
                                  sem).wait()

    rows = pl.ds(s * ROWS_PER_TILE, ROWS_PER_TILE)
    pltpu.sync_copy(h2c.at[rows], h_sp.at[rows])
    pltpu.sync_copy(h2c.at[rows], acc.at[rows])
    plsc.subcore_barrier()

    for p in range(N_PHASES):
        pchunks = pl.ds(p * CPP, CPP)
        pltpu.sync_copy(src3.at[s].at[pchunks], src_v)
        pltpu.sync_copy(dst3.at[s].at[pchunks], dst_v)

        for i in range(K):
            pltpu.async_copy(h_sp.at[src_v.at[i]], bufs.at[i], gsA)
        for i in range(K):
            pltpu.async_copy(bufs.at[K + i], acc.at[pl.ds(N_NODES, CHUNK)],
                             ssB)

        def step(t, _):
            g0 = 2 * t
            g1 = g0 + 1
            gw = lax.rem(g0 + 2, CPP)

            drain(ssB, K)
            for i in range(K):
                pltpu.async_copy(h_sp.at[src_v.at[g1 * K + i]],
                                 bufs.at[K + i], gsB)
            drain(gsA, K)
            for i in range(K):
                pltpu.async_copy(bufs.at[i], acc.at[dst_v.at[g0 * K + i]],
                                 ssA, add=True)
            drain(ssA, K)
            for i in range(K):
                pltpu.async_copy(h_sp.at[src_v.at[gw * K + i]], bufs.at[i],
                                 gsA)
            drain(gsB, K)
            for i in range(K):
                pltpu.async_copy(bufs.at[K + i],
                                 acc.at[dst_v.at[g1 * K + i]], ssB, add=True)
            return _

        lax.fori_loop(0, NT, step, 0, unroll=False)
        drain(ssB, K)
        drain(gsA, K)

    plsc.subcore_barrier()

    pltpu.sync_copy(acc.at[rows], z_out.at[c].at[rows])


def _make_sc_segsum():
    mesh = plsc.VectorSubcoreMesh(core_axis_name="c", subcore_axis_name="s")
    return pl.kernel(
        _sc_body,
        out_type=jax.ShapeDtypeStruct((2, N_NODES, HALF), jnp.float32),
        mesh=mesh,
        compiler_params=pltpu.CompilerParams(use_tc_tiling_on_sc=False,
                                             skip_device_barrier=True),
        scratch_types=[
            pltpu.VMEM_SHARED((N_NODES, HALF), jnp.float32),
            pltpu.VMEM_SHARED((ACC_ROWS, HALF), jnp.float32),
            pltpu.VMEM((CPP, CHUNK), jnp.int32),
            pltpu.VMEM((CPP, CHUNK), jnp.int32),
            pltpu.VMEM((2 * K, CHUNK, HALF), jnp.float32),
            pltpu.SemaphoreType.DMA,
            pltpu.SemaphoreType.DMA,
            pltpu.SemaphoreType.DMA,
            pltpu.SemaphoreType.DMA,
        ],
    )


def _dense_layer(z2, h2, w1, b1, w2, b2, gam, bet):
    zin = jnp.concatenate([z2[0], z2[1]], axis=-1)
    t = jnp.dot(zin, w1[...], preferred_element_type=jnp.float32) + b1[...]
    t = jnp.maximum(t, 0.0)
    t = jnp.dot(t, w2[...], preferred_element_type=jnp.float32) + b2[...]
    mu = jnp.mean(t, axis=0, keepdims=True)
    var = jnp.mean((t - mu) ** 2, axis=0, keepdims=True)
    t = (t - mu) * lax.rsqrt(var + BN_EPS) * gam[...] + bet[...]
    t = jnp.maximum(t, 0.0)
    return jnp.concatenate([h2[0], h2[1]], axis=-1) + t


def _tc_layer_body(z2, h2, w1, b1, w2, b2, gam, bet, out):
    hnew = _dense_layer(z2, h2, w1, b1, w2, b2, gam, bet)
    out[0] = hnew[:, :HALF]
    out[1] = hnew[:, HALF:]


def _tc_layer(z2, h2, p):
    return pl.pallas_call(
        _tc_layer_body,
        out_shape=jax.ShapeDtypeStruct((2, N_NODES, HALF), jnp.float32),
    )(z2, h2, p["W1"], p["b1"][None, :], p["W2"], p["b2"][None, :],
      p["gamma"][None, :], p["beta"][None, :])


def _tc_last_body(z2, h2, w1, b1, w2, b2, gam, bet, batch,
                  wm1, bm1, wm2, bm2, out):
    h = _dense_layer(z2, h2, w1, b1, w2, b2, gam, bet)
    seg = lax.broadcasted_iota(jnp.int32, (N_GRAPHS, N_NODES), 0)
    onehot = (seg == batch[...]).astype(jnp.float32)
    sums = jnp.dot(onehot, h, preferred_element_type=jnp.float32)
    counts = jnp.sum(onehot, axis=1, keepdims=True)
    g = sums / jnp.maximum(counts, 1.0)
    g = jnp.maximum(
        jnp.dot(g, wm1[...], preferred_element_type=jnp.float32) + bm1[...],
        0.0)
    out[...] = jnp.dot(g, wm2[...], preferred_element_type=jnp.float32) \
        + bm2[...]


def _tc_last(z2, h2, p, batch2d, params):
    return pl.pallas_call(
        _tc_last_body,
        out_shape=jax.ShapeDtypeStruct((N_GRAPHS, N_CLASSES), jnp.float32),
    )(z2, h2, p["W1"], p["b1"][None, :], p["W2"], p["b2"][None, :],
      p["gamma"][None, :], p["beta"][None, :], batch2d,
      params["Wm1"], params["bm1"][None, :],
      params["Wm2"], params["bm2"][None, :])


def kernel(x, edge_index, batch, params):
    n_pad = EDGES_PER_TILE_PAD - EDGES_PER_TILE
    src3 = jnp.concatenate(
        [edge_index[0].reshape(N_TILES, EDGES_PER_TILE),
         jnp.zeros((N_TILES, n_pad), jnp.int32)],
        axis=1).reshape(N_TILES, N_CHUNKS, CHUNK)
    dst3 = jnp.concatenate(
        [edge_index[1].reshape(N_TILES, EDGES_PER_TILE),
         jnp.full((N_TILES, n_pad), N_NODES, jnp.int32)],
        axis=1).reshape(N_TILES, N_CHUNKS, CHUNK)
    h2 = x.reshape(N_NODES, 2, HALF).transpose(1, 0, 2)
    sc_segsum = _make_sc_segsum()
    for l in range(N_LAYERS - 1):
        z2 = sc_segsum(h2, src3, dst3)
        h2 = _tc_layer(z2, h2, params[f"layer{l}"])
    z2 = sc_segsum(h2, src3, dst3)
    return _tc_last(z2, h2, params[f"layer{N_LAYERS - 1}"],
                    batch[None, :], params)


# speedup vs baseline: 2.0409x; 1.4652x over previous
import jax
import jax.numpy as jnp
from jax import lax
from jax.experimental import pallas as pl
from jax.experimental.pallas import tpu as pltpu, tpu_sc as plsc

N_NODES = 10000
N_EDGES = 320000
D_IN = 128
HID = 128
N_CLASSES = 10
N_GRAPHS = 64
N_LAYERS = 3
BN_EPS = 1e-5

HALF = HID // 2
N_TILES = 16
EDGES_PER_TILE = N_EDGES // N_TILES
CHUNK = 80
K = 4
N_CHUNKS = 256
EDGES_PER_TILE_PAD = N_CHUNKS * CHUNK
ROWS_PER_TILE = N_NODES // N_TILES
NT = N_CHUNKS // (2 * K)
ACC_ROWS = N_NODES + CHUNK


def _sc_body(h2, src3, dst3, z_out, h_sp, acc, src_v, dst_v, bufs,
             gsA, gsB, ssA, ssB):
    c = lax.axis_index("c")
    s = lax.axis_index("s")
    h2c = h2.at[c]

    def drain(sem, n):
        for _ in range(n):
            pltpu.make_async_copy(h2c.at[pl.ds(0, CHUNK)], bufs.at[0],
                                  sem).wait()

    pltpu.sync_copy(src3.at[s], src_v)
    pltpu.sync_copy(dst3.at[s], dst_v)

    rows = pl.ds(s * ROWS_PER_TILE, ROWS_PER_TILE)
    pltpu.sync_copy(h2c.at[rows], h_sp.at[rows])
    pltpu.sync_copy(h2c.at[rows], acc.at[rows])
    plsc.subcore_barrier()

    for i in range(K):
        pltpu.async_copy(h_sp.at[src_v.at[i]], bufs.at[i], gsA)
    for i in range(K):
        pltpu.async_copy(bufs.at[K + i], acc.at[pl.ds(N_NODES, CHUNK)], ssB)

    def step(t, _):
        g0 = 2 * t
        g1 = g0 + 1
        gw = lax.rem(g0 + 2, N_CHUNKS)

        drain(ssB, K)
        for i in range(K):
            pltpu.async_copy(h_sp.at[src_v.at[g1 * K + i]], bufs.at[K + i],
                             gsB)
        drain(gsA, K)
        for i in range(K):
            pltpu.async_copy(bufs.at[i], acc.at[dst_v.at[g0 * K + i]], ssA,
                             add=True)
        drain(ssA, K)
        for i in range(K):
            pltpu.async_copy(h_sp.at[src_v.at[gw * K + i]], bufs.at[i], gsA)
        drain(gsB, K)
        for i in range(K):
            pltpu.async_copy(bufs.at[K + i], acc.at[dst_v.at[g1 * K + i]],
                             ssB, add=True)
        return _

    lax.fori_loop(0, NT, step, 0, unroll=False)
    drain(ssB, K)
    drain(gsA, K)
    plsc.subcore_barrier()

    pltpu.sync_copy(acc.at[rows], z_out.at[c].at[rows])


def _make_sc_segsum():
    mesh = plsc.VectorSubcoreMesh(core_axis_name="c", subcore_axis_name="s")
    return pl.kernel(
        _sc_body,
        out_type=jax.ShapeDtypeStruct((2, N_NODES, HALF), jnp.bfloat16),
        mesh=mesh,
        compiler_params=pltpu.CompilerParams(use_tc_tiling_on_sc=False),
        scratch_types=[
            pltpu.VMEM_SHARED((N_NODES, HALF), jnp.bfloat16),
            pltpu.VMEM_SHARED((ACC_ROWS, HALF), jnp.bfloat16),
            pltpu.VMEM((N_CHUNKS, CHUNK), jnp.int32),
            pltpu.VMEM((N_CHUNKS, CHUNK), jnp.int32),
            pltpu.VMEM((2 * K, CHUNK, HALF), jnp.bfloat16),
            pltpu.SemaphoreType.DMA,
            pltpu.SemaphoreType.DMA,
            pltpu.SemaphoreType.DMA,
            pltpu.SemaphoreType.DMA,
        ],
    )


def _dense_layer(z2, h2, w1, b1, w2, b2, gam, bet):
    hin = jnp.concatenate([h2[0], h2[1]], axis=-1)
    zb = jnp.concatenate([z2[0], z2[1]], axis=-1).astype(jnp.float32)
    hb = hin.astype(jnp.bfloat16).astype(jnp.float32)
    zin = hin + (zb - hb)
    t = jnp.dot(zin, w1[...], preferred_element_type=jnp.float32) + b1[...]
    t = jnp.maximum(t, 0.0)
    t = jnp.dot(t, w2[...], preferred_element_type=jnp.float32) + b2[...]
    mu = jnp.mean(t, axis=0, keepdims=True)
    var = jnp.mean((t - mu) ** 2, axis=0, keepdims=True)
    t = (t - mu) * lax.rsqrt(var + BN_EPS) * gam[...] + bet[...]
    t = jnp.maximum(t, 0.0)
    return hin + t


def _tc_layer_body(z2, h2, w1, b1, w2, b2, gam, bet, out, outb):
    hnew = _dense_layer(z2, h2, w1, b1, w2, b2, gam, bet)
    out[0] = hnew[:, :HALF]
    out[1] = hnew[:, HALF:]
    hb = hnew.astype(jnp.bfloat16)
    outb[0] = hb[:, :HALF]
    outb[1] = hb[:, HALF:]


def _tc_layer(z2, h2, p):
    return pl.pallas_call(
        _tc_layer_body,
        out_shape=(jax.ShapeDtypeStruct((2, N_NODES, HALF), jnp.float32),
                   jax.ShapeDtypeStruct((2, N_NODES, HALF), jnp.bfloat16)),
    )(z2, h2, p["W1"], p["b1"][None, :], p["W2"], p["b2"][None, :],
      p["gamma"][None, :], p["beta"][None, :])


def _tc_last_body(z2, h2, w1, b1, w2, b2, gam, bet, batch,
                  wm1, bm1, wm2, bm2, out):
    h = _dense_layer(z2, h2, w1, b1, w2, b2, gam, bet)
    seg = lax.broadcasted_iota(jnp.int32, (N_GRAPHS, N_NODES), 0)
    onehot = (seg == batch[...]).astype(jnp.float32)
    sums = jnp.dot(onehot, h, preferred_element_type=jnp.float32)
    counts = jnp.sum(onehot, axis=1, keepdims=True)
    g = sums / jnp.maximum(counts, 1.0)
    g = jnp.maximum(
        jnp.dot(g, wm1[...], preferred_element_type=jnp.float32) + bm1[...],
        0.0)
    out[...] = jnp.dot(g, wm2[...], preferred_element_type=jnp.float32) \
        + bm2[...]


def _tc_last(z2, h2, p, batch2d, params):
    return pl.pallas_call(
        _tc_last_body,
        out_shape=jax.ShapeDtypeStruct((N_GRAPHS, N_CLASSES), jnp.float32),
    )(z2, h2, p["W1"], p["b1"][None, :], p["W2"], p["b2"][None, :],
      p["gamma"][None, :], p["beta"][None, :], batch2d,
      params["Wm1"], params["bm1"][None, :],
      params["Wm2"], params["bm2"][None, :])


def kernel(x, edge_index, batch, params):
    n_pad = EDGES_PER_TILE_PAD - EDGES_PER_TILE
    src3 = jnp.concatenate(
        [edge_index[0].reshape(N_TILES, EDGES_PER_TILE),
         jnp.zeros((N_TILES, n_pad), jnp.int32)],
        axis=1).reshape(N_TILES, N_CHUNKS, CHUNK)
    dst3 = jnp.concatenate(
        [edge_index[1].reshape(N_TILES, EDGES_PER_TILE),
         jnp.full((N_TILES, n_pad), N_NODES, jnp.int32)],
        axis=1).reshape(N_TILES, N_CHUNKS, CHUNK)
    h2 = x.reshape(N_NODES, 2, HALF).transpose(1, 0, 2)
    h2b = h2.astype(jnp.bfloat16)
    sc_segsum = _make_sc_segsum()
    for l in range(N_LAYERS - 1):
        z2 = sc_segsum(h2b, src3, dst3)
        h2, h2b = _tc_layer(z2, h2, params[f"layer{l}"])
    z2 = sc_segsum(h2b, src3, dst3)
    return _tc_last(z2, h2, params[f"layer{N_LAYERS - 1}"],
                    batch[None, :], params)
